# baseline (device time: 86402 ns/iter reference)
import jax
import jax.numpy as jnp
from jax import lax
from jax.experimental import pallas as pl
from jax.experimental.pallas import tpu as pltpu

NZ = 4
ROWS = 1024
COLS = 512
DR, DC = 8, 128


def kernel(x, dest):
    dest2d = dest.reshape(DR, DC)

    def body(x_ref, d_ref, xg_ref, dg_ref, send_x, recv_x, send_d, recv_d):
        my_x = lax.axis_index("x")
        my_y = lax.axis_index("y")
        mz = lax.axis_index("z")

        barrier = pltpu.get_barrier_semaphore()
        for k in range(1, NZ):
            pl.semaphore_signal(
                barrier, inc=1,
                device_id=(my_x, my_y, (mz + k) % NZ),
                device_id_type=pl.DeviceIdType.MESH,
            )
        pl.semaphore_wait(barrier, NZ - 1)

        xg_ref[mz] = x_ref[...]
        dg_ref[mz] = d_ref[...]

        rdmas = []
        for k in range(1, NZ):
            tgt = (my_x, my_y, (mz + k) % NZ)
            rx = pltpu.make_async_remote_copy(
                src_ref=x_ref,
                dst_ref=xg_ref.at[mz],
                send_sem=send_x.at[k - 1],
                recv_sem=recv_x.at[k - 1],
                device_id=tgt,
                device_id_type=pl.DeviceIdType.MESH,
            )
            rd = pltpu.make_async_remote_copy(
                src_ref=d_ref,
                dst_ref=dg_ref.at[mz],
                send_sem=send_d.at[k - 1],
                recv_sem=recv_d.at[k - 1],
                device_id=tgt,
                device_id_type=pl.DeviceIdType.MESH,
            )
            rx.start()
            rd.start()
            rdmas.append((rx, rd))

        for rx, rd in rdmas:
            rx.wait()
            rd.wait()

    xg, dg = pl.pallas_call(
        body,
        out_shape=[
            jax.ShapeDtypeStruct((NZ, ROWS, COLS), jnp.float32),
            jax.ShapeDtypeStruct((NZ, DR, DC), jnp.int32),
        ],
        in_specs=[
            pl.BlockSpec(memory_space=pltpu.VMEM),
            pl.BlockSpec(memory_space=pltpu.VMEM),
        ],
        out_specs=[
            pl.BlockSpec(memory_space=pltpu.VMEM),
            pl.BlockSpec(memory_space=pltpu.VMEM),
        ],
        scratch_shapes=[
            pltpu.SemaphoreType.DMA((NZ - 1,)),
            pltpu.SemaphoreType.DMA((NZ - 1,)),
            pltpu.SemaphoreType.DMA((NZ - 1,)),
            pltpu.SemaphoreType.DMA((NZ - 1,)),
        ],
        compiler_params=pltpu.CompilerParams(collective_id=0),
    )(x, dest2d)

    xfull = xg.reshape(NZ * ROWS, COLS)
    dfull = dg.reshape(NZ * ROWS)

    mz = lax.axis_index("z")
    order = jnp.argsort(dfull, stable=True)
    mine = lax.dynamic_slice(order, (mz * ROWS,), (ROWS,))
    return jnp.take(xfull, mine, axis=0)
